# Initial kernel scaffold; baseline (speedup 1.0000x reference)
import jax
import jax.numpy as jnp
from jax.experimental import pallas as pl


def kernel(input):
    raise NotImplementedError("write your pallas kernel here")


# speedup vs baseline: 17.1240x; 17.1240x over previous
import functools

import jax
import jax.numpy as jnp
from jax import lax
from jax.experimental import pallas as pl
from jax.experimental.pallas import tpu as pltpu
from jax.experimental.pallas import tpu_sc as plsc

ROWS = 64
N = 32768
L = 16
NCH = N // L
NUM_WORKERS = 32
ROWS_PER_WORKER = ROWS // NUM_WORKERS

BISECT_ITERS = 24
MICHELOT_ITERS = 2

_mesh = plsc.VectorSubcoreMesh(core_axis_name="c", subcore_axis_name="s")


def _vmax16(v):
    s = [v[i] for i in range(L)]
    while len(s) > 1:
        s = [jnp.maximum(s[i], s[i + 1]) for i in range(0, len(s), 2)]
    return s[0]


def _vsum16(v):
    s = [v[i] for i in range(L)]
    while len(s) > 1:
        s = [s[i] + s[i + 1] for i in range(0, len(s), 2)]
    return s[0]


@functools.partial(
    pl.kernel,
    out_type=jax.ShapeDtypeStruct((ROWS, N), jnp.float32),
    mesh=_mesh,
    compiler_params=pltpu.CompilerParams(needs_layout_passes=False),
    scratch_types=[
        pltpu.VMEM((N,), jnp.float32),
        pltpu.VMEM((N + L,), jnp.float32),
    ],
)
def _sparsemax_sc(x_hbm, out_hbm, row_v, cand_v):
    nc = _mesh.num_cores
    wid = lax.axis_index("s") * nc + lax.axis_index("c")

    def do_row(row):
        pltpu.sync_copy(x_hbm.at[row], row_v)

        ninf = jnp.full((L,), -jnp.inf, jnp.float32)

        def mx_body(i, accs):
            a0, a1, a2, a3 = accs
            b = i * (4 * L)
            a0 = jnp.maximum(a0, row_v[pl.ds(b, L)])
            a1 = jnp.maximum(a1, row_v[pl.ds(b + L, L)])
            a2 = jnp.maximum(a2, row_v[pl.ds(b + 2 * L, L)])
            a3 = jnp.maximum(a3, row_v[pl.ds(b + 3 * L, L)])
            return (a0, a1, a2, a3)

        a0, a1, a2, a3 = lax.fori_loop(0, NCH // 4, mx_body,
                                       (ninf, ninf, ninf, ninf))
        m = _vmax16(jnp.maximum(jnp.maximum(a0, a1), jnp.maximum(a2, a3)))
        thresh = m - 1.0

        def cbody(i, c):
            v = row_v[pl.ds(i * L, L)]
            msk = v > thresh
            pos = jnp.cumsum(msk.astype(jnp.int32))
            idx = c + pos - 1
            plsc.store_scatter(cand_v, [idx], v, mask=msk)
            return c + pos[L - 1]

        c = lax.fori_loop(0, NCH, cbody, jnp.int32(0))

        pad_idx = c + lax.iota(jnp.int32, L)
        plsc.store_scatter(cand_v, [pad_idx],
                           jnp.full((L,), thresh - 1.0, jnp.float32))
        nchunks = lax.shift_right_logical(c + (L - 1), 4)

        def bis(_, carry):
            lo, hi = carry
            mid = 0.5 * (lo + hi)

            def sbody(i, acc):
                v = cand_v[pl.ds(i * L, L)]
                return acc + jnp.maximum(v - mid, 0.0)

            acc = lax.fori_loop(0, nchunks, sbody,
                                jnp.zeros((L,), jnp.float32))
            ge = _vsum16(acc) >= 1.0
            return (jnp.where(ge, mid, lo), jnp.where(ge, hi, mid))

        lo, _ = lax.fori_loop(0, BISECT_ITERS, bis, (thresh, m))

        def mich(_, tau):
            def nb(i, carry):
                kacc, sacc = carry
                v = cand_v[pl.ds(i * L, L)]
                msk = v > tau
                kacc = kacc + msk.astype(jnp.float32)
                sacc = sacc + jnp.where(msk, v, 0.0)
                return (kacc, sacc)

            kacc, sacc = lax.fori_loop(
                0, nchunks, nb,
                (jnp.zeros((L,), jnp.float32), jnp.zeros((L,), jnp.float32)))
            num = jnp.full((L,), _vsum16(sacc) - 1.0, jnp.float32)
            den = jnp.full((L,), _vsum16(kacc), jnp.float32)
            return (num / den)[0]

        tau = lax.fori_loop(0, MICHELOT_ITERS, mich, lo)

        def obody(i, _):
            b = i * (4 * L)
            for j in range(4):
                v = row_v[pl.ds(b + j * L, L)]
                row_v[pl.ds(b + j * L, L)] = jnp.maximum(v - tau, 0.0)
            return 0

        lax.fori_loop(0, NCH // 4, obody, 0)
        pltpu.sync_copy(row_v, out_hbm.at[row])

    for r in range(ROWS_PER_WORKER):
        do_row(wid * ROWS_PER_WORKER + r)


def kernel(input):
    return _sparsemax_sc(input)


# speedup vs baseline: 22.3473x; 1.3050x over previous
import functools

import jax
import jax.numpy as jnp
from jax import lax
from jax.experimental import pallas as pl
from jax.experimental.pallas import tpu as pltpu
from jax.experimental.pallas import tpu_sc as plsc

ROWS = 64
N = 32768
L = 16
NCH = N // L
NUM_WORKERS = 32
ROWS_PER_WORKER = ROWS // NUM_WORKERS

BISECT_ITERS = 24
MICHELOT_ITERS = 2

_mesh = plsc.VectorSubcoreMesh(core_axis_name="c", subcore_axis_name="s")


def _vred16(v, op):
    s = [v[i] for i in range(L)]
    while len(s) > 1:
        s = [op(s[i], s[i + 1]) for i in range(0, len(s), 2)]
    return s[0]


def _vmax16(v):
    return _vred16(v, jnp.maximum)


def _vsum16(v):
    return _vred16(v, lambda a, b: a + b)


@functools.partial(
    pl.kernel,
    out_type=jax.ShapeDtypeStruct((ROWS, N), jnp.float32),
    mesh=_mesh,
    compiler_params=pltpu.CompilerParams(needs_layout_passes=False),
    scratch_types=[
        pltpu.VMEM((N,), jnp.float32),
        pltpu.VMEM((N,), jnp.float32),
    ],
)
def _sparsemax_sc(x_hbm, out_hbm, row_v, cand_v):
    nc = _mesh.num_cores
    wid = lax.axis_index("s") * nc + lax.axis_index("c")
    iota = lax.iota(jnp.int32, L)

    def do_row(row):
        pltpu.sync_copy(x_hbm.at[row], row_v)

        ninf = jnp.full((L,), -jnp.inf, jnp.float32)

        def mx_body(i, accs):
            a0, a1, a2, a3 = accs
            b = i * (4 * L)
            a0 = jnp.maximum(a0, row_v[pl.ds(b, L)])
            a1 = jnp.maximum(a1, row_v[pl.ds(b + L, L)])
            a2 = jnp.maximum(a2, row_v[pl.ds(b + 2 * L, L)])
            a3 = jnp.maximum(a3, row_v[pl.ds(b + 3 * L, L)])
            return (a0, a1, a2, a3)

        a0, a1, a2, a3 = lax.fori_loop(0, NCH // 4, mx_body,
                                       (ninf, ninf, ninf, ninf))
        m = _vmax16(jnp.maximum(jnp.maximum(a0, a1), jnp.maximum(a2, a3)))
        thresh = m - 1.0

        def cbody(i, cnt):
            b = i * (4 * L)
            for j in range(4):
                v = row_v[pl.ds(b + j * L, L)]
                msk = v > thresh
                idx = lax.shift_left(cnt, 4) + iota
                plsc.store_scatter(cand_v, [idx], v, mask=msk)
                cnt = cnt + msk.astype(jnp.int32)
            return cnt

        cnt = lax.fori_loop(0, NCH // 4, cbody, jnp.zeros((L,), jnp.int32))
        maxcnt = _vmax16(cnt)

        def fbody(k, _):
            v = cand_v[pl.ds(k * L, L)]
            cand_v[pl.ds(k * L, L)] = jnp.where(cnt > k, v, thresh - 1.0)
            return 0

        lax.fori_loop(0, maxcnt, fbody, 0)

        def bis(_, carry):
            lo, hi = carry
            mid = 0.5 * (lo + hi)

            def sbody(i, acc):
                v = cand_v[pl.ds(i * L, L)]
                return acc + jnp.maximum(v - mid, 0.0)

            acc = lax.fori_loop(0, maxcnt, sbody,
                                jnp.zeros((L,), jnp.float32))
            ge = _vsum16(acc) >= 1.0
            return (jnp.where(ge, mid, lo), jnp.where(ge, hi, mid))

        lo, _ = lax.fori_loop(0, BISECT_ITERS, bis, (thresh, m))

        def mich(_, tau):
            def nb(i, carry):
                kacc, sacc = carry
                v = cand_v[pl.ds(i * L, L)]
                msk = v > tau
                kacc = kacc + msk.astype(jnp.float32)
                sacc = sacc + jnp.where(msk, v, 0.0)
                return (kacc, sacc)

            kacc, sacc = lax.fori_loop(
                0, maxcnt, nb,
                (jnp.zeros((L,), jnp.float32), jnp.zeros((L,), jnp.float32)))
            num = jnp.full((L,), _vsum16(sacc) - 1.0, jnp.float32)
            den = jnp.full((L,), _vsum16(kacc), jnp.float32)
            return (num / den)[0]

        tau = lax.fori_loop(0, MICHELOT_ITERS, mich, lo)

        def obody(i, _):
            b = i * (4 * L)
            for j in range(4):
                v = row_v[pl.ds(b + j * L, L)]
                row_v[pl.ds(b + j * L, L)] = jnp.maximum(v - tau, 0.0)
            return 0

        lax.fori_loop(0, NCH // 4, obody, 0)
        pltpu.sync_copy(row_v, out_hbm.at[row])

    for r in range(ROWS_PER_WORKER):
        do_row(wid * ROWS_PER_WORKER + r)


def kernel(input):
    return _sparsemax_sc(input)


# speedup vs baseline: 37.9720x; 1.6992x over previous
import functools

import jax
import jax.numpy as jnp
from jax import lax
from jax.experimental import pallas as pl
from jax.experimental.pallas import tpu as pltpu
from jax.experimental.pallas import tpu_sc as plsc

ROWS = 64
N = 32768
L = 16
NCH = N // L
NUM_WORKERS = 32
ROWS_PER_WORKER = ROWS // NUM_WORKERS

BISECT_ITERS = 24
MICHELOT_ITERS = 2

_mesh = plsc.VectorSubcoreMesh(core_axis_name="c", subcore_axis_name="s")


def _vred16(v, op):
    s = [v[i] for i in range(L)]
    while len(s) > 1:
        s = [op(s[i], s[i + 1]) for i in range(0, len(s), 2)]
    return s[0]


def _vmax16(v):
    return _vred16(v, jnp.maximum)


def _vsum16(v):
    return _vred16(v, lambda a, b: a + b)


@functools.partial(
    pl.kernel,
    out_type=jax.ShapeDtypeStruct((ROWS, N), jnp.float32),
    mesh=_mesh,
    compiler_params=pltpu.CompilerParams(needs_layout_passes=False),
    scratch_types=[
        pltpu.VMEM((N,), jnp.float32),
        pltpu.VMEM((N,), jnp.float32),
    ],
)
def _sparsemax_sc(x_hbm, out_hbm, row_v, cand_v):
    nc = _mesh.num_cores
    wid = lax.axis_index("s") * nc + lax.axis_index("c")
    iota = lax.iota(jnp.int32, L)

    def do_row(row):
        pltpu.sync_copy(x_hbm.at[row], row_v)

        ninf = jnp.full((L,), -jnp.inf, jnp.float32)

        @plsc.parallel_loop(0, NCH, 2, unroll=4, carry=(ninf, ninf))
        def mx_accs(i, accs):
            a0, a1 = accs
            b = i * L
            a0 = jnp.maximum(a0, row_v[pl.ds(b, L)])
            a1 = jnp.maximum(a1, row_v[pl.ds(b + L, L)])
            return (a0, a1)

        m = _vmax16(jnp.maximum(mx_accs[0], mx_accs[1]))
        thresh = m - 1.0

        @plsc.parallel_loop(0, NCH, unroll=8, carry=jnp.zeros((L,), jnp.int32))
        def cnt(i, cnt):
            v = row_v[pl.ds(i * L, L)]
            msk = v > thresh
            idx = lax.shift_left(cnt, 4) + iota
            plsc.store_scatter(cand_v, [idx], v, mask=msk)
            return cnt + msk.astype(jnp.int32)

        maxcnt = _vmax16(cnt)

        def fbody(k, _):
            v = cand_v[pl.ds(k * L, L)]
            cand_v[pl.ds(k * L, L)] = jnp.where(cnt > k, v, thresh - 1.0)
            return 0

        lax.fori_loop(0, maxcnt, fbody, 0)

        def bis(_, carry):
            lo, hi = carry
            mid = 0.5 * (lo + hi)

            def sbody(i, acc):
                v = cand_v[pl.ds(i * L, L)]
                return acc + jnp.maximum(v - mid, 0.0)

            acc = lax.fori_loop(0, maxcnt, sbody,
                                jnp.zeros((L,), jnp.float32))
            ge = _vsum16(acc) >= 1.0
            return (jnp.where(ge, mid, lo), jnp.where(ge, hi, mid))

        lo, _ = lax.fori_loop(0, BISECT_ITERS, bis, (thresh, m))

        def mich(_, tau):
            def nb(i, carry):
                kacc, sacc = carry
                v = cand_v[pl.ds(i * L, L)]
                msk = v > tau
                kacc = kacc + msk.astype(jnp.float32)
                sacc = sacc + jnp.where(msk, v, 0.0)
                return (kacc, sacc)

            kacc, sacc = lax.fori_loop(
                0, maxcnt, nb,
                (jnp.zeros((L,), jnp.float32), jnp.zeros((L,), jnp.float32)))
            num = jnp.full((L,), _vsum16(sacc) - 1.0, jnp.float32)
            den = jnp.full((L,), _vsum16(kacc), jnp.float32)
            return (num / den)[0]

        tau = lax.fori_loop(0, MICHELOT_ITERS, mich, lo)

        @plsc.parallel_loop(0, NCH, unroll=8)
        def _(i):
            b = i * L
            v = row_v[pl.ds(b, L)]
            row_v[pl.ds(b, L)] = jnp.maximum(v - tau, 0.0)

        pltpu.sync_copy(row_v, out_hbm.at[row])

    for r in range(ROWS_PER_WORKER):
        do_row(wid * ROWS_PER_WORKER + r)


def kernel(input):
    return _sparsemax_sc(input)


# speedup vs baseline: 40.4430x; 1.0651x over previous
import functools

import jax
import jax.numpy as jnp
from jax import lax
from jax.experimental import pallas as pl
from jax.experimental.pallas import tpu as pltpu
from jax.experimental.pallas import tpu_sc as plsc

ROWS = 64
N = 32768
L = 16
NCH = N // L
NUM_WORKERS = 32
ROWS_PER_WORKER = ROWS // NUM_WORKERS

BISECT_ITERS = 24
MICHELOT_ITERS = 2

_mesh = plsc.VectorSubcoreMesh(core_axis_name="c", subcore_axis_name="s")


def _vred16(v, op):
    s = [v[i] for i in range(L)]
    while len(s) > 1:
        s = [op(s[i], s[i + 1]) for i in range(0, len(s), 2)]
    return s[0]


def _vmax16(v):
    return _vred16(v, jnp.maximum)


def _vsum16(v):
    return _vred16(v, lambda a, b: a + b)


@functools.partial(
    pl.kernel,
    out_type=jax.ShapeDtypeStruct((ROWS, N), jnp.float32),
    mesh=_mesh,
    compiler_params=pltpu.CompilerParams(needs_layout_passes=False),
    scratch_types=[
        pltpu.VMEM((N,), jnp.float32),
        pltpu.VMEM((N,), jnp.float32),
        pltpu.VMEM((N,), jnp.float32),
        pltpu.SemaphoreType.DMA,
        pltpu.SemaphoreType.DMA,
        pltpu.SemaphoreType.DMA,
        pltpu.SemaphoreType.DMA,
    ],
)
def _sparsemax_sc(x_hbm, out_hbm, rowa_v, rowb_v, cand_v, si0, si1, so0, so1):
    nc = _mesh.num_cores
    wid = lax.axis_index("s") * nc + lax.axis_index("c")
    iota = lax.iota(jnp.int32, L)

    def compute_tau(row_v):
        ninf = jnp.full((L,), -jnp.inf, jnp.float32)

        @plsc.parallel_loop(0, NCH, 2, unroll=4, carry=(ninf, ninf))
        def mx_accs(i, accs):
            a0, a1 = accs
            b = i * L
            a0 = jnp.maximum(a0, row_v[pl.ds(b, L)])
            a1 = jnp.maximum(a1, row_v[pl.ds(b + L, L)])
            return (a0, a1)

        m = _vmax16(jnp.maximum(mx_accs[0], mx_accs[1]))
        thresh = m - 1.0

        @plsc.parallel_loop(0, NCH, unroll=8, carry=jnp.zeros((L,), jnp.int32))
        def cnt(i, cnt):
            v = row_v[pl.ds(i * L, L)]
            msk = v > thresh
            idx = lax.shift_left(cnt, 4) + iota
            plsc.store_scatter(cand_v, [idx], v, mask=msk)
            return cnt + msk.astype(jnp.int32)

        maxcnt = _vmax16(cnt)

        def fbody(k, _):
            v = cand_v[pl.ds(k * L, L)]
            cand_v[pl.ds(k * L, L)] = jnp.where(cnt > k, v, thresh - 1.0)
            return 0

        lax.fori_loop(0, maxcnt, fbody, 0)

        def bis(_, carry):
            lo, hi = carry
            mid = 0.5 * (lo + hi)

            def sbody(i, acc):
                v = cand_v[pl.ds(i * L, L)]
                return acc + jnp.maximum(v - mid, 0.0)

            acc = lax.fori_loop(0, maxcnt, sbody,
                                jnp.zeros((L,), jnp.float32))
            ge = _vsum16(acc) >= 1.0
            return (jnp.where(ge, mid, lo), jnp.where(ge, hi, mid))

        lo, _ = lax.fori_loop(0, BISECT_ITERS, bis, (thresh, m))

        def mich(_, tau):
            def nb(i, carry):
                kacc, sacc = carry
                v = cand_v[pl.ds(i * L, L)]
                msk = v > tau
                kacc = kacc + msk.astype(jnp.float32)
                sacc = sacc + jnp.where(msk, v, 0.0)
                return (kacc, sacc)

            kacc, sacc = lax.fori_loop(
                0, maxcnt, nb,
                (jnp.zeros((L,), jnp.float32), jnp.zeros((L,), jnp.float32)))
            num = jnp.full((L,), _vsum16(sacc) - 1.0, jnp.float32)
            den = jnp.full((L,), _vsum16(kacc), jnp.float32)
            return (num / den)[0]

        return lax.fori_loop(0, MICHELOT_ITERS, mich, lo)

    def relu_pass(row_v, tau):
        @plsc.parallel_loop(0, NCH, unroll=8)
        def _(i):
            b = i * L
            v = row_v[pl.ds(b, L)]
            row_v[pl.ds(b, L)] = jnp.maximum(v - tau, 0.0)

    r0 = wid * ROWS_PER_WORKER
    in0 = pltpu.async_copy(x_hbm.at[r0], rowa_v, si0)
    in1 = pltpu.async_copy(x_hbm.at[r0 + 1], rowb_v, si1)

    in0.wait()
    tau0 = compute_tau(rowa_v)
    relu_pass(rowa_v, tau0)
    out0 = pltpu.async_copy(rowa_v, out_hbm.at[r0], so0)

    in1.wait()
    tau1 = compute_tau(rowb_v)
    relu_pass(rowb_v, tau1)
    out1 = pltpu.async_copy(rowb_v, out_hbm.at[r0 + 1], so1)

    out0.wait()
    out1.wait()


def kernel(input):
    return _sparsemax_sc(input)
